# baseline (speedup 1.0000x reference)
import functools

import numpy as np
import jax
import jax.numpy as jnp
from jax import lax
from jax.experimental import pallas as pl
from jax.experimental.pallas import tpu as pltpu
from jax.experimental.pallas import tpu_sc as plsc

_VSZ = 1000001
_DSZ = 64
_B = 4096
_L = 200

_SC_PT = 11776
_SC_ROWS = 32 * _SC_PT
_SC_START = 1000000 - _SC_ROWS
_SC_CH = 368
_SC_NCHK = _SC_PT // _SC_CH

_ROW_BLK = 8192
_NSTREAM = 4
_NSTEP = -(-_SC_START // (_ROW_BLK * _NSTREAM))

_NW = 32
_BPW = _B // _NW
_TPW = _BPW * _L
_GCHUNK = 128
_NCH = _TPW // _GCHUNK

_DN = lax.GatherDimensionNumbers(offset_dims=(), collapsed_slice_dims=(0,),
                                 start_index_map=(0,))


def _lane_sum(p, perms):
    for perm in perms:
        p = p + lax.gather(p, perm, _DN, slice_sizes=(1,),
                           mode=lax.GatherScatterMode.PROMISE_IN_BOUNDS)
    return p


def _proj_body(lut0, lut1, lut2, lut3, w_ref, pd0, pd1, pd2, pd3):
    w = w_ref[...]
    wd = w[1:2, :] - w[0:1, :]
    for x_ref, o_ref in ((lut0, pd0), (lut1, pd1), (lut2, pd2), (lut3, pd3)):
        pd = lax.dot_general(wd, x_ref[...], (((1,), (1,)), ((), ())),
                             preferred_element_type=jnp.float32)
        o_ref[...] = pd.reshape(1, 1, _ROW_BLK)


def _project_table(lut_weight, out_weight):
    lut_spec = [
        pl.BlockSpec((_ROW_BLK, _DSZ),
                     functools.partial(lambda k, i: (_NSTEP * k + i, 0), k))
        for k in range(_NSTREAM)
    ]
    outs = pl.pallas_call(
        _proj_body,
        grid=(_NSTEP,),
        in_specs=lut_spec + [pl.BlockSpec((2, _DSZ), lambda i: (0, 0))],
        out_specs=[pl.BlockSpec((1, 1, _ROW_BLK), lambda i: (i, 0, 0))]
        * _NSTREAM,
        out_shape=[jax.ShapeDtypeStruct((_NSTEP, 1, _ROW_BLK), jnp.float32)]
        * _NSTREAM,
    )(lut_weight, lut_weight, lut_weight, lut_weight, out_weight)
    return jnp.concatenate([o.reshape(-1) for o in outs])


def _sc_scan_body(lut_hbm, w_hbm, pd_hbm, wv, buf_a, buf_b, out_v,
                  sem_a, sem_b):
    c = lax.axis_index("c")
    s = lax.axis_index("s")
    wid = s * 2 + c
    base = _SC_START + wid * _SC_PT

    pltpu.sync_copy(w_hbm, wv)
    wd = [wv[1, pl.ds(q * 16, 16)] - wv[0, pl.ds(q * 16, 16)]
          for q in range(4)]
    lane = lax.iota(jnp.int32, 16)
    perms = [jnp.reshape(lane ^ sh, (16, 1)) for sh in (8, 4, 2, 1)]

    bufs = (buf_a, buf_b)
    sems = (sem_a, sem_b)

    def issue(ci, u):
        pltpu.async_copy(lut_hbm.at[pl.ds(base + ci * _SC_CH, _SC_CH)],
                         bufs[u], sems[u])

    def wait(u):
        pltpu.make_async_copy(lut_hbm.at[pl.ds(0, _SC_CH)], bufs[u],
                              sems[u]).wait()

    def process(ci, u):
        buf = bufs[u]

        def group(i, carry):
            res = jnp.zeros((16,), jnp.float32)
            for uu in range(16):
                r = i * 16 + uu
                p = (buf[r, pl.ds(0, 16)] * wd[0]
                     + buf[r, pl.ds(16, 16)] * wd[1]
                     + buf[r, pl.ds(32, 16)] * wd[2]
                     + buf[r, pl.ds(48, 16)] * wd[3])
                p = _lane_sum(p, perms)
                res = jnp.where(lane == uu, p, res)
            out_v[pl.ds(ci * _SC_CH + i * 16, 16)] = res
            return carry

        lax.fori_loop(0, _SC_CH // 16, group, 0, unroll=False)

    issue(0, 0)
    issue(1, 1)

    def chunk_pair(g, carry):
        for u in range(2):
            ci = g * 2 + u
            wait(u)
            process(ci, u)

            @pl.when(ci + 2 < _SC_NCHK)
            def _():
                issue(ci + 2, u)

        return carry

    lax.fori_loop(0, _SC_NCHK // 2, chunk_pair, 0, unroll=False)
    pltpu.sync_copy(out_v, pd_hbm.at[pl.ds(wid * _SC_PT, _SC_PT)])


def _sc_scan(lut_weight, out_weight):
    mesh = plsc.VectorSubcoreMesh(core_axis_name="c", subcore_axis_name="s")
    run = pl.kernel(
        _sc_scan_body,
        out_type=jax.ShapeDtypeStruct((_SC_ROWS,), jnp.float32),
        mesh=mesh,
        scratch_types=[
            pltpu.VMEM((2, _DSZ), jnp.float32),
            pltpu.VMEM((_SC_CH, _DSZ), jnp.float32),
            pltpu.VMEM((_SC_CH, _DSZ), jnp.float32),
            pltpu.VMEM((_SC_PT,), jnp.float32),
            pltpu.SemaphoreType.DMA,
            pltpu.SemaphoreType.DMA,
        ],
    )
    return run(lut_weight, out_weight)


def _sc_body(pd_hbm, ids_hbm, d_hbm, idx_v, vals_v, out_v, sem):
    c = lax.axis_index("c")
    s = lax.axis_index("s")
    wid = s * 2 + c

    pltpu.sync_copy(ids_hbm.at[wid], idx_v)

    nacc = _BPW // 16

    def issue(j, carry):
        pltpu.async_copy(
            pd_hbm.at[idx_v.at[j]],
            vals_v.at[pl.ds(j * _GCHUNK, _GCHUNK)],
            sem)
        return carry

    lax.fori_loop(0, _NCH, issue, 0, unroll=False)
    pltpu.make_async_copy(pd_hbm.at[pl.ds(0, _TPW)], vals_v, sem).wait()

    def acc_chunk(j, accs):
        base = j * _GCHUNK
        return tuple(
            accs[r] + vals_v[pl.ds(base + r * 16, 16)]
            for r in range(nacc))

    accs = lax.fori_loop(
        0, _NCH, acc_chunk,
        tuple(jnp.zeros((16,), jnp.float32) for _ in range(nacc)),
        unroll=False)

    for r in range(nacc):
        out_v[pl.ds(r * 16, 16)] = accs[r] * (1.0 / _L)

    pltpu.sync_copy(out_v, d_hbm.at[pl.ds(wid * _BPW, _BPW)])


def _sc_gather_mean(pd_flat, ids3):
    mesh = plsc.VectorSubcoreMesh(core_axis_name="c", subcore_axis_name="s")
    run = pl.kernel(
        _sc_body,
        out_type=jax.ShapeDtypeStruct((_B,), jnp.float32),
        mesh=mesh,
        scratch_types=[
            pltpu.VMEM((_NCH, _GCHUNK), jnp.int32),
            pltpu.VMEM((_TPW,), jnp.float32),
            pltpu.VMEM((_BPW,), jnp.float32),
            pltpu.SemaphoreType.DMA,
        ],
    )
    return run(pd_flat, ids3)


def _head_body(d_ref, b_ref, o0_ref, o1_ref):
    delta = d_ref[...] + (b_ref[1] - b_ref[0])
    sp = jnp.maximum(delta, 0.0) + jnp.log1p(jnp.exp(-jnp.abs(delta)))
    o0_ref[...] = -sp
    o1_ref[...] = delta - sp


def _head(d2, out_bias):
    return pl.pallas_call(
        _head_body,
        in_specs=[pl.BlockSpec((_NW, _BPW), lambda: (0, 0)),
                  pl.BlockSpec(memory_space=pltpu.SMEM)],
        out_specs=[pl.BlockSpec((_NW, _BPW), lambda: (0, 0))] * 2,
        out_shape=[jax.ShapeDtypeStruct((_NW, _BPW), jnp.float32)] * 2,
    )(d2, out_bias)


def kernel(input, lut_weight, out_weight, out_bias):
    ids = input.astype(jnp.int32)
    pd_tc = _project_table(lut_weight, out_weight)
    pd_sc = _sc_scan(lut_weight, out_weight)
    pd = jnp.concatenate([pd_tc[:_SC_START], pd_sc])
    ids3 = ids.reshape(_NW, _BPW, _L).transpose(0, 2, 1)
    delta = _sc_gather_mean(pd, ids3)
    o0, o1 = _head(delta.reshape(_NW, _BPW), out_bias)
    return jnp.stack([o0.reshape(_B), o1.reshape(_B)], axis=-1)
